# baseline (speedup 1.0000x reference)
import functools

import jax
import jax.numpy as jnp
from jax import lax
from jax.experimental import pallas as pl
from jax.experimental.pallas import tpu as pltpu
from jax.experimental.pallas import tpu_sc as plsc

N = 50000
E = 800000
L = 4
D = 64

EB = 4000
NB = 5000
NT = 32
SUB = 125
MAC = 1000
NPAD = 51200

_mesh = plsc.VectorSubcoreMesh(core_axis_name="c", subcore_axis_name="s")
_f32 = jnp.float32
_bf16 = jnp.bfloat16
_sc_params = pltpu.CompilerParams(use_tc_tiling_on_sc=False)


@functools.partial(
    pl.kernel,
    out_type=[jax.ShapeDtypeStruct((E, D), _bf16),
              jax.ShapeDtypeStruct((E, D), _bf16)],
    mesh=_mesh,
    scratch_types=[pltpu.VMEM((8, SUB), jnp.int32),
                   pltpu.VMEM((MAC, D), _bf16),
                   pltpu.SemaphoreType.DMA],
    compiler_params=_sc_params,
)
def _sc_gather(h_hbm, dst_hbm, src_hbm, hd_hbm, hs_hbm, idx_v, rows_v, sem):
    c = lax.axis_index("c")
    s = lax.axis_index("s")
    wid = s * 2 + c
    base = wid * (E // NT)

    @pl.loop(0, (E // NT) // MAC)
    def _(g):
        off = base + g * MAC
        row0 = pl.multiple_of(off // SUB, 8)
        for ind_hbm, out_hbm in ((dst_hbm, hd_hbm), (src_hbm, hs_hbm)):
            pltpu.sync_copy(ind_hbm.at[pl.ds(row0, 8)], idx_v)
            cps = [
                pltpu.async_copy(h_hbm.at[idx_v.at[j]],
                                 rows_v.at[pl.ds(j * SUB, SUB)], sem)
                for j in range(8)
            ]
            for cp in cps:
                cp.wait()
            pltpu.sync_copy(rows_v, out_hbm.at[pl.ds(off, MAC)])


@functools.partial(
    pl.kernel,
    out_type=jax.ShapeDtypeStruct((NPAD, D), _f32),
    mesh=_mesh,
    scratch_types=[pltpu.VMEM((8, SUB), jnp.int32),
                   pltpu.VMEM((MAC, D // 4), _f32),
                   pltpu.VMEM((4, 16), _f32),
                   pltpu.VMEM((4, 16), _f32),
                   pltpu.VMEM_SHARED((NPAD, D // 4), _f32),
                   pltpu.SemaphoreType.DMA],
    compiler_params=_sc_params,
)
def _sc_scatter(z_hbm, dst_hbm, zero_hbm, sc_hbm, sh_hbm, aggr_hbm,
                idx_v, mbuf_v, sc_v, sh_v, acc_sh, sem):
    c = lax.axis_index("c")
    s = lax.axis_index("s")
    rows_per_sub = NPAD // 16
    rbase = s * rows_per_sub
    ebase = s * (E // 16)

    pltpu.sync_copy(sc_hbm, sc_v)
    pltpu.sync_copy(sh_hbm, sh_v)

    for cc in range(2):
        colbase = c * (D // 2) + cc * (D // 4)
        kk = c * 2 + cc
        sv = sc_v[kk]
        tv = sh_v[kk]

        pltpu.sync_copy(zero_hbm, acc_sh.at[pl.ds(rbase, rows_per_sub)])
        plsc.subcore_barrier()

        @pl.loop(0, (E // 16) // MAC)
        def _(g):
            off = ebase + g * MAC
            row0 = pl.multiple_of(off // SUB, 8)
            pltpu.sync_copy(dst_hbm.at[pl.ds(row0, 8)], idx_v)
            pltpu.sync_copy(z_hbm.at[pl.ds(off, MAC), pl.ds(colbase, D // 4)],
                            mbuf_v)

            @plsc.parallel_loop(0, MAC, unroll=8)
            def _(r):
                mbuf_v[r] = jnp.maximum(mbuf_v[r] * sv + tv, 0.0)

            cps = [
                pltpu.async_copy(mbuf_v.at[pl.ds(j * SUB, SUB)],
                                 acc_sh.at[idx_v.at[j]], sem, add=True)
                for j in range(8)
            ]
            for cp in cps:
                cp.wait()

        plsc.subcore_barrier()
        pltpu.sync_copy(acc_sh.at[pl.ds(rbase, rows_per_sub)],
                        aggr_hbm.at[pl.ds(rbase, rows_per_sub),
                                    pl.ds(colbase, D // 4)])
        plsc.subcore_barrier()


def _rspec(blk):
    return pl.BlockSpec(blk, lambda i: (i, 0))


def _wspec(shape):
    return pl.BlockSpec(shape, lambda i: (0, 0))


def _accum_stats(st_ref, z):
    @pl.when(pl.program_id(0) == 0)
    def _():
        st_ref[...] = jnp.zeros_like(st_ref)

    st_ref[0:1, :] += jnp.sum(z, axis=0, keepdims=True)
    st_ref[1:2, :] += jnp.sum(z * z, axis=0, keepdims=True)


def _in_proj_body(x_ref, w_ref, b_ref, o_ref, ob_ref):
    h = (jnp.dot(x_ref[...], w_ref[...],
                 preferred_element_type=_f32) + b_ref[...])
    o_ref[...] = h
    ob_ref[...] = h.astype(_bf16)


def _edge1_body(hd, hs, ea, wa, wb, wc, b, z_ref, st_ref):
    z = (jnp.dot(hd[...], wa[...], preferred_element_type=_f32)
         + jnp.dot(hs[...], wb[...], preferred_element_type=_f32)
         + jnp.dot(ea[...], wc[...], preferred_element_type=_f32)
         + b[...])
    z_ref[...] = z
    _accum_stats(st_ref, z)


def _mlp2_body(z1, s1, t1, w, b, z_ref, st_ref):
    a = jnp.maximum(z1[...] * s1[...] + t1[...], 0.0)
    z = jnp.dot(a, w[...], preferred_element_type=_f32) + b[...]
    z_ref[...] = z
    _accum_stats(st_ref, z)


def _node1_body(h, ag, wa, wb, b, z_ref, st_ref):
    z = (jnp.dot(h[...], wa[...], preferred_element_type=_f32)
         + jnp.dot(ag[...], wb[...], preferred_element_type=_f32)
         + b[...])
    z_ref[...] = z
    _accum_stats(st_ref, z)


def _node3_body(zu2, s, t, h, o_ref, ob_ref):
    hn = h[...] + jnp.maximum(zu2[...] * s[...] + t[...], 0.0)
    o_ref[...] = hn
    ob_ref[...] = hn.astype(_bf16)


def _node3_head_body(zu2, s, t, h, wpm, bpm, wpo, bpo, o_ref):
    hn = h[...] + jnp.maximum(zu2[...] * s[...] + t[...], 0.0)
    hp = jnp.maximum(jnp.dot(hn, wpm[...], preferred_element_type=_f32)
                     + bpm[...], 0.0)
    o_ref[...] = (jnp.dot(hp, wpo[...], preferred_element_type=_f32)
                  + bpo[...])


def _stats_shape():
    return jax.ShapeDtypeStruct((8, D), _f32)


def _bn_coeffs(st, g, be, n):
    mean = st[0] / n
    var = st[1] / n - mean * mean
    a = g * lax.rsqrt(var + 1e-5)
    t = be - mean * a
    return a.reshape(1, D), t.reshape(1, D)


def kernel(x, edge_index, edge_attr, batch, W_in, b_in, Wm1, bm1, gm1, bem1,
           Wm2, bm2, gm2, bem2, Wu1, bu1, gu1, beu1, Wu2, bu2, gu2, beu2,
           W_rm, b_rm, W_ro, b_ro, W_pm, b_pm, W_po, b_po):
    del batch, W_rm, b_rm, W_ro, b_ro

    src2 = edge_index[0].reshape(E // SUB, SUB)
    dst2 = edge_index[1].reshape(E // SUB, SUB)
    zero_init = jnp.zeros((NPAD // 16, D // 4), _f32)

    egrid = (E // EB,)
    ngrid = (N // NB,)

    h, hb = pl.pallas_call(
        _in_proj_body,
        grid=ngrid,
        in_specs=[_rspec((NB, 128)), _wspec((128, D)), _wspec((1, D))],
        out_specs=[_rspec((NB, D)), _rspec((NB, D))],
        out_shape=[jax.ShapeDtypeStruct((N, D), _f32),
                   jax.ShapeDtypeStruct((N, D), _bf16)],
    )(x, W_in, b_in.reshape(1, D))

    for l in range(L):
        hd, hs = _sc_gather(hb, dst2, src2)

        z1, st1 = pl.pallas_call(
            _edge1_body,
            grid=egrid,
            in_specs=[_rspec((EB, D)), _rspec((EB, D)), _rspec((EB, 16)),
                      _wspec((D, D)), _wspec((D, D)), _wspec((16, D)),
                      _wspec((1, D))],
            out_specs=[_rspec((EB, D)), _wspec((8, D))],
            out_shape=[jax.ShapeDtypeStruct((E, D), _f32), _stats_shape()],
        )(hd, hs, edge_attr, Wm1[l, :D].astype(_bf16),
          Wm1[l, D:2 * D].astype(_bf16), Wm1[l, 2 * D:],
          bm1[l].reshape(1, D))
        s1, t1 = _bn_coeffs(st1, gm1[l], bem1[l], E)

        z2, st2 = pl.pallas_call(
            _mlp2_body,
            grid=egrid,
            in_specs=[_rspec((EB, D)), _wspec((1, D)), _wspec((1, D)),
                      _wspec((D, D)), _wspec((1, D))],
            out_specs=[_rspec((EB, D)), _wspec((8, D))],
            out_shape=[jax.ShapeDtypeStruct((E, D), _f32), _stats_shape()],
        )(z1, s1, t1, Wm2[l], bm2[l].reshape(1, D))
        s2, t2 = _bn_coeffs(st2, gm2[l], bem2[l], E)

        aggr = _sc_scatter(z2, dst2, zero_init,
                           s2.reshape(4, 16), t2.reshape(4, 16))

        zu1, su1 = pl.pallas_call(
            _node1_body,
            grid=ngrid,
            in_specs=[_rspec((NB, D)), _rspec((NB, D)), _wspec((D, D)),
                      _wspec((D, D)), _wspec((1, D))],
            out_specs=[_rspec((NB, D)), _wspec((8, D))],
            out_shape=[jax.ShapeDtypeStruct((N, D), _f32), _stats_shape()],
        )(h, aggr, Wu1[l, :D], Wu1[l, D:], bu1[l].reshape(1, D))
        a1, b1 = _bn_coeffs(su1, gu1[l], beu1[l], N)

        zu2, su2 = pl.pallas_call(
            _mlp2_body,
            grid=ngrid,
            in_specs=[_rspec((NB, D)), _wspec((1, D)), _wspec((1, D)),
                      _wspec((D, D)), _wspec((1, D))],
            out_specs=[_rspec((NB, D)), _wspec((8, D))],
            out_shape=[jax.ShapeDtypeStruct((N, D), _f32), _stats_shape()],
        )(zu1, a1, b1, Wu2[l], bu2[l].reshape(1, D))
        a2, b2 = _bn_coeffs(su2, gu2[l], beu2[l], N)

        if l < L - 1:
            h, hb = pl.pallas_call(
                _node3_body,
                grid=ngrid,
                in_specs=[_rspec((NB, D)), _wspec((1, D)), _wspec((1, D)),
                          _rspec((NB, D))],
                out_specs=[_rspec((NB, D)), _rspec((NB, D))],
                out_shape=[jax.ShapeDtypeStruct((N, D), _f32),
                           jax.ShapeDtypeStruct((N, D), _bf16)],
            )(zu2, a2, b2, h)
        else:
            x_out = pl.pallas_call(
                _node3_head_body,
                grid=ngrid,
                in_specs=[_rspec((NB, D)), _wspec((1, D)), _wspec((1, D)),
                          _rspec((NB, D)), _wspec((D, D)), _wspec((1, D)),
                          _wspec((D, D)), _wspec((1, D))],
                out_specs=_rspec((NB, D)),
                out_shape=jax.ShapeDtypeStruct((N, D), _f32),
            )(zu2, a2, b2, h, W_pm, b_pm.reshape(1, D),
              W_po, b_po.reshape(1, D))
    return x_out


# speedup vs baseline: 1.0978x; 1.0978x over previous
import functools

import jax
import jax.numpy as jnp
from jax import lax
from jax.experimental import pallas as pl
from jax.experimental.pallas import tpu as pltpu
from jax.experimental.pallas import tpu_sc as plsc

N = 50000
E = 800000
L = 4
D = 64

EB = 4000
NB = 5000
NT = 32
SUB = 125
MAC = 1000
NPAD = 51200

_mesh = plsc.VectorSubcoreMesh(core_axis_name="c", subcore_axis_name="s")
_f32 = jnp.float32
_bf16 = jnp.bfloat16
_sc_params = pltpu.CompilerParams(use_tc_tiling_on_sc=False)


@functools.partial(
    pl.kernel,
    out_type=[jax.ShapeDtypeStruct((E, D), _f32),
              jax.ShapeDtypeStruct((E, D), _f32)],
    mesh=_mesh,
    scratch_types=[pltpu.VMEM((8, SUB), jnp.int32),
                   pltpu.VMEM((MAC, D), _f32),
                   pltpu.SemaphoreType.DMA],
    compiler_params=_sc_params,
)
def _sc_gather(h_hbm, dst_hbm, src_hbm, hd_hbm, hs_hbm, idx_v, rows_v, sem):
    c = lax.axis_index("c")
    s = lax.axis_index("s")
    wid = s * 2 + c
    base = wid * (E // NT)

    @pl.loop(0, (E // NT) // MAC)
    def _(g):
        off = base + g * MAC
        row0 = pl.multiple_of(off // SUB, 8)
        for ind_hbm, out_hbm in ((dst_hbm, hd_hbm), (src_hbm, hs_hbm)):
            pltpu.sync_copy(ind_hbm.at[pl.ds(row0, 8)], idx_v)
            cps = [
                pltpu.async_copy(h_hbm.at[idx_v.at[j]],
                                 rows_v.at[pl.ds(j * SUB, SUB)], sem)
                for j in range(8)
            ]
            for cp in cps:
                cp.wait()
            pltpu.sync_copy(rows_v, out_hbm.at[pl.ds(off, MAC)])


@functools.partial(
    pl.kernel,
    out_type=jax.ShapeDtypeStruct((NPAD, D), _f32),
    mesh=_mesh,
    scratch_types=[pltpu.VMEM((8, SUB), jnp.int32),
                   pltpu.VMEM((MAC, D // 4), _f32),
                   pltpu.VMEM((4, 16), _f32),
                   pltpu.VMEM((4, 16), _f32),
                   pltpu.VMEM_SHARED((NPAD, D // 4), _f32),
                   pltpu.SemaphoreType.DMA],
    compiler_params=_sc_params,
)
def _sc_scatter(z_hbm, dst_hbm, zero_hbm, sc_hbm, sh_hbm, aggr_hbm,
                idx_v, mbuf_v, sc_v, sh_v, acc_sh, sem):
    c = lax.axis_index("c")
    s = lax.axis_index("s")
    rows_per_sub = NPAD // 16
    rbase = s * rows_per_sub
    ebase = s * (E // 16)

    pltpu.sync_copy(sc_hbm, sc_v)
    pltpu.sync_copy(sh_hbm, sh_v)

    for cc in range(2):
        colbase = c * (D // 2) + cc * (D // 4)
        kk = c * 2 + cc
        sv = sc_v[kk]
        tv = sh_v[kk]

        pltpu.sync_copy(zero_hbm, acc_sh.at[pl.ds(rbase, rows_per_sub)])
        plsc.subcore_barrier()

        @pl.loop(0, (E // 16) // MAC)
        def _(g):
            off = ebase + g * MAC
            row0 = pl.multiple_of(off // SUB, 8)
            pltpu.sync_copy(dst_hbm.at[pl.ds(row0, 8)], idx_v)
            pltpu.sync_copy(z_hbm.at[pl.ds(off, MAC), pl.ds(colbase, D // 4)],
                            mbuf_v)

            @plsc.parallel_loop(0, MAC, unroll=8)
            def _(r):
                mbuf_v[r] = jnp.maximum(mbuf_v[r] * sv + tv, 0.0)

            cps = [
                pltpu.async_copy(mbuf_v.at[pl.ds(j * SUB, SUB)],
                                 acc_sh.at[idx_v.at[j]], sem, add=True)
                for j in range(8)
            ]
            for cp in cps:
                cp.wait()

        plsc.subcore_barrier()
        pltpu.sync_copy(acc_sh.at[pl.ds(rbase, rows_per_sub)],
                        aggr_hbm.at[pl.ds(rbase, rows_per_sub),
                                    pl.ds(colbase, D // 4)])
        plsc.subcore_barrier()


def _rspec(blk):
    return pl.BlockSpec(blk, lambda i: (i, 0))


def _wspec(shape):
    return pl.BlockSpec(shape, lambda i: (0, 0))


def _accum_stats(st_ref, z):
    @pl.when(pl.program_id(0) == 0)
    def _():
        st_ref[...] = jnp.zeros_like(st_ref)

    st_ref[0:1, :] += jnp.sum(z, axis=0, keepdims=True)
    st_ref[1:2, :] += jnp.sum(z * z, axis=0, keepdims=True)


def _in_proj_body(x_ref, w_ref, b_ref, o_ref):
    o_ref[...] = (jnp.dot(x_ref[...], w_ref[...],
                          preferred_element_type=_f32) + b_ref[...])


def _bdot(a, w):
    return jnp.dot(a.astype(_bf16), w.astype(_bf16),
                   preferred_element_type=_f32)


def _edge1_body(hd, hs, ea, wa, wb, wc, b, z_ref, st_ref):
    z = (_bdot(hd[...], wa[...]) + _bdot(hs[...], wb[...])
         + _bdot(ea[...], wc[...]) + b[...])
    z_ref[...] = z
    _accum_stats(st_ref, z)


def _mlp2_body(z1, s1, t1, w, b, z_ref, st_ref):
    a = jnp.maximum(z1[...] * s1[...] + t1[...], 0.0)
    z = _bdot(a, w[...]) + b[...]
    z_ref[...] = z
    _accum_stats(st_ref, z)


def _node1_body(h, ag, wa, wb, b, z_ref, st_ref):
    z = _bdot(h[...], wa[...]) + _bdot(ag[...], wb[...]) + b[...]
    z_ref[...] = z
    _accum_stats(st_ref, z)


def _node3_body(zu2, s, t, h, o_ref):
    o_ref[...] = h[...] + jnp.maximum(zu2[...] * s[...] + t[...], 0.0)


def _node3_head_body(zu2, s, t, h, wpm, bpm, wpo, bpo, o_ref):
    hn = h[...] + jnp.maximum(zu2[...] * s[...] + t[...], 0.0)
    hp = jnp.maximum(jnp.dot(hn, wpm[...], preferred_element_type=_f32)
                     + bpm[...], 0.0)
    o_ref[...] = (jnp.dot(hp, wpo[...], preferred_element_type=_f32)
                  + bpo[...])


def _stats_shape():
    return jax.ShapeDtypeStruct((8, D), _f32)


def _bn_coeffs(st, g, be, n):
    mean = st[0] / n
    var = st[1] / n - mean * mean
    a = g * lax.rsqrt(var + 1e-5)
    t = be - mean * a
    return a.reshape(1, D), t.reshape(1, D)


def kernel(x, edge_index, edge_attr, batch, W_in, b_in, Wm1, bm1, gm1, bem1,
           Wm2, bm2, gm2, bem2, Wu1, bu1, gu1, beu1, Wu2, bu2, gu2, beu2,
           W_rm, b_rm, W_ro, b_ro, W_pm, b_pm, W_po, b_po):
    del batch, W_rm, b_rm, W_ro, b_ro

    src2 = edge_index[0].reshape(E // SUB, SUB)
    dst2 = edge_index[1].reshape(E // SUB, SUB)
    zero_init = jnp.zeros((NPAD // 16, D // 4), _f32)

    egrid = (E // EB,)
    ngrid = (N // NB,)

    h = pl.pallas_call(
        _in_proj_body,
        grid=ngrid,
        in_specs=[_rspec((NB, 128)), _wspec((128, D)), _wspec((1, D))],
        out_specs=_rspec((NB, D)),
        out_shape=jax.ShapeDtypeStruct((N, D), _f32),
    )(x, W_in, b_in.reshape(1, D))

    for l in range(L):
        hd, hs = _sc_gather(h, dst2, src2)

        z1, st1 = pl.pallas_call(
            _edge1_body,
            grid=egrid,
            in_specs=[_rspec((EB, D)), _rspec((EB, D)), _rspec((EB, 16)),
                      _wspec((D, D)), _wspec((D, D)), _wspec((16, D)),
                      _wspec((1, D))],
            out_specs=[_rspec((EB, D)), _wspec((8, D))],
            out_shape=[jax.ShapeDtypeStruct((E, D), _f32), _stats_shape()],
        )(hd, hs, edge_attr, Wm1[l, :D], Wm1[l, D:2 * D], Wm1[l, 2 * D:],
          bm1[l].reshape(1, D))
        s1, t1 = _bn_coeffs(st1, gm1[l], bem1[l], E)

        z2, st2 = pl.pallas_call(
            _mlp2_body,
            grid=egrid,
            in_specs=[_rspec((EB, D)), _wspec((1, D)), _wspec((1, D)),
                      _wspec((D, D)), _wspec((1, D))],
            out_specs=[_rspec((EB, D)), _wspec((8, D))],
            out_shape=[jax.ShapeDtypeStruct((E, D), _f32), _stats_shape()],
        )(z1, s1, t1, Wm2[l], bm2[l].reshape(1, D))
        s2, t2 = _bn_coeffs(st2, gm2[l], bem2[l], E)

        aggr = _sc_scatter(z2, dst2, zero_init,
                           s2.reshape(4, 16), t2.reshape(4, 16))

        zu1, su1 = pl.pallas_call(
            _node1_body,
            grid=ngrid,
            in_specs=[_rspec((NB, D)), _rspec((NB, D)), _wspec((D, D)),
                      _wspec((D, D)), _wspec((1, D))],
            out_specs=[_rspec((NB, D)), _wspec((8, D))],
            out_shape=[jax.ShapeDtypeStruct((N, D), _f32), _stats_shape()],
        )(h, aggr, Wu1[l, :D], Wu1[l, D:], bu1[l].reshape(1, D))
        a1, b1 = _bn_coeffs(su1, gu1[l], beu1[l], N)

        zu2, su2 = pl.pallas_call(
            _mlp2_body,
            grid=ngrid,
            in_specs=[_rspec((NB, D)), _wspec((1, D)), _wspec((1, D)),
                      _wspec((D, D)), _wspec((1, D))],
            out_specs=[_rspec((NB, D)), _wspec((8, D))],
            out_shape=[jax.ShapeDtypeStruct((N, D), _f32), _stats_shape()],
        )(zu1, a1, b1, Wu2[l], bu2[l].reshape(1, D))
        a2, b2 = _bn_coeffs(su2, gu2[l], beu2[l], N)

        if l < L - 1:
            h = pl.pallas_call(
                _node3_body,
                grid=ngrid,
                in_specs=[_rspec((NB, D)), _wspec((1, D)), _wspec((1, D)),
                          _rspec((NB, D))],
                out_specs=_rspec((NB, D)),
                out_shape=jax.ShapeDtypeStruct((N, D), _f32),
            )(zu2, a2, b2, h)
        else:
            x_out = pl.pallas_call(
                _node3_head_body,
                grid=ngrid,
                in_specs=[_rspec((NB, D)), _wspec((1, D)), _wspec((1, D)),
                          _rspec((NB, D)), _wspec((D, D)), _wspec((1, D)),
                          _wspec((D, D)), _wspec((1, D))],
                out_specs=_rspec((NB, D)),
                out_shape=jax.ShapeDtypeStruct((N, D), _f32),
            )(zu2, a2, b2, h, W_pm, b_pm.reshape(1, D),
              W_po, b_po.reshape(1, D))
    return x_out


# speedup vs baseline: 1.1343x; 1.0332x over previous
import functools

import jax
import jax.numpy as jnp
from jax import lax
from jax.experimental import pallas as pl
from jax.experimental.pallas import tpu as pltpu
from jax.experimental.pallas import tpu_sc as plsc

N = 50000
E = 800000
L = 4
D = 64

EB = 4000
NB = 5000
NT = 32
SUB = 125
MAC = 1000
NPAD = 51200

HA = 416000
HB = 384000

_mesh = plsc.VectorSubcoreMesh(core_axis_name="c", subcore_axis_name="s")
_f32 = jnp.float32
_bf16 = jnp.bfloat16
_sc_params = pltpu.CompilerParams(use_tc_tiling_on_sc=False)


def _make_gather(ne):
    per_tile = ne // NT

    @functools.partial(
        pl.kernel,
        out_type=[jax.ShapeDtypeStruct((ne, D), _f32),
                  jax.ShapeDtypeStruct((ne, D), _f32)],
        mesh=_mesh,
        scratch_types=[pltpu.VMEM((8, SUB), jnp.int32),
                       pltpu.VMEM((MAC, D), _f32),
                       pltpu.SemaphoreType.DMA],
        compiler_params=_sc_params,
    )
    def gather(h_hbm, dst_hbm, src_hbm, hd_hbm, hs_hbm, idx_v, rows_v, sem):
        c = lax.axis_index("c")
        s = lax.axis_index("s")
        wid = s * 2 + c
        base = wid * per_tile

        @pl.loop(0, per_tile // MAC)
        def _(g):
            off = base + g * MAC
            row0 = pl.multiple_of(off // SUB, 8)
            for ind_hbm, out_hbm in ((dst_hbm, hd_hbm), (src_hbm, hs_hbm)):
                pltpu.sync_copy(ind_hbm.at[pl.ds(row0, 8)], idx_v)
                cps = [
                    pltpu.async_copy(h_hbm.at[idx_v.at[j]],
                                     rows_v.at[pl.ds(j * SUB, SUB)], sem)
                    for j in range(8)
                ]
                for cp in cps:
                    cp.wait()
                pltpu.sync_copy(rows_v, out_hbm.at[pl.ds(off, MAC)])

    return gather


_gather_a = _make_gather(HA)
_gather_b = _make_gather(HB)


def _make_scatter(ne):
    per_sub = ne // 16

    @functools.partial(
        pl.kernel,
        out_type=jax.ShapeDtypeStruct((NPAD, D), _f32),
        mesh=_mesh,
        scratch_types=[pltpu.VMEM((8, SUB), jnp.int32),
                       pltpu.VMEM((MAC, D // 4), _f32),
                       pltpu.VMEM((4, 16), _f32),
                       pltpu.VMEM((4, 16), _f32),
                       pltpu.VMEM_SHARED((NPAD, D // 4), _f32),
                       pltpu.SemaphoreType.DMA],
        compiler_params=_sc_params,
    )
    def scatter(z_hbm, dst_hbm, zero_hbm, sc_hbm, sh_hbm, aggr_hbm,
                idx_v, mbuf_v, sc_v, sh_v, acc_sh, sem):
        c = lax.axis_index("c")
        s = lax.axis_index("s")
        rows_per_sub = NPAD // 16
        rbase = s * rows_per_sub
        ebase = s * per_sub

        pltpu.sync_copy(sc_hbm, sc_v)
        pltpu.sync_copy(sh_hbm, sh_v)

        for cc in range(2):
            colbase = c * (D // 2) + cc * (D // 4)
            kk = c * 2 + cc
            sv = sc_v[kk]
            tv = sh_v[kk]

            pltpu.sync_copy(zero_hbm, acc_sh.at[pl.ds(rbase, rows_per_sub)])
            plsc.subcore_barrier()

            @pl.loop(0, per_sub // MAC)
            def _(g):
                off = ebase + g * MAC
                row0 = pl.multiple_of(off // SUB, 8)
                pltpu.sync_copy(dst_hbm.at[pl.ds(row0, 8)], idx_v)
                pltpu.sync_copy(z_hbm.at[pl.ds(off, MAC),
                                         pl.ds(colbase, D // 4)], mbuf_v)

                @plsc.parallel_loop(0, MAC, unroll=8)
                def _(r):
                    mbuf_v[r] = jnp.maximum(mbuf_v[r] * sv + tv, 0.0)

                cps = [
                    pltpu.async_copy(mbuf_v.at[pl.ds(j * SUB, SUB)],
                                     acc_sh.at[idx_v.at[j]], sem, add=True)
                    for j in range(8)
                ]
                for cp in cps:
                    cp.wait()

            plsc.subcore_barrier()
            pltpu.sync_copy(acc_sh.at[pl.ds(rbase, rows_per_sub)],
                            aggr_hbm.at[pl.ds(rbase, rows_per_sub),
                                        pl.ds(colbase, D // 4)])
            plsc.subcore_barrier()

    return scatter


_scatter_a = _make_scatter(HA)
_scatter_b = _make_scatter(HB)


def _rspec(blk):
    return pl.BlockSpec(blk, lambda i: (i, 0))


def _wspec(shape):
    return pl.BlockSpec(shape, lambda i: (0, 0))


def _accum_stats(st_ref, z):
    @pl.when(pl.program_id(0) == 0)
    def _():
        st_ref[...] = jnp.zeros_like(st_ref)

    st_ref[0:1, :] += jnp.sum(z, axis=0, keepdims=True)
    st_ref[1:2, :] += jnp.sum(z * z, axis=0, keepdims=True)


def _in_proj_body(x_ref, w_ref, b_ref, o_ref):
    o_ref[...] = (jnp.dot(x_ref[...], w_ref[...],
                          preferred_element_type=_f32) + b_ref[...])


def _bdot(a, w):
    return jnp.dot(a.astype(_bf16), w.astype(_bf16),
                   preferred_element_type=_f32)


def _edge1_body(hd, hs, ea, wa, wb, wc, b, z_ref, st_ref):
    z = (_bdot(hd[...], wa[...]) + _bdot(hs[...], wb[...])
         + _bdot(ea[...], wc[...]) + b[...])
    z_ref[...] = z
    _accum_stats(st_ref, z)


def _mlp2_body(z1, s1, t1, w, b, z_ref, st_ref):
    a = jnp.maximum(z1[...] * s1[...] + t1[...], 0.0)
    z = _bdot(a, w[...]) + b[...]
    z_ref[...] = z
    _accum_stats(st_ref, z)


def _nodeA_body(h, wa, b, o_ref):
    o_ref[...] = _bdot(h[...], wa[...]) + b[...]


def _node1_body(p, aga, agb, wb, z_ref, st_ref):
    z = p[...] + _bdot(aga[...] + agb[...], wb[...])
    z_ref[...] = z
    _accum_stats(st_ref, z)


def _node3_body(zu2, s, t, h, o_ref):
    o_ref[...] = h[...] + jnp.maximum(zu2[...] * s[...] + t[...], 0.0)


def _node3_head_body(zu2, s, t, h, wpm, bpm, wpo, bpo, o_ref):
    hn = h[...] + jnp.maximum(zu2[...] * s[...] + t[...], 0.0)
    hp = jnp.maximum(jnp.dot(hn, wpm[...], preferred_element_type=_f32)
                     + bpm[...], 0.0)
    o_ref[...] = (jnp.dot(hp, wpo[...], preferred_element_type=_f32)
                  + bpo[...])


def _stats_shape():
    return jax.ShapeDtypeStruct((8, D), _f32)


def _bn_coeffs(st, g, be, n):
    mean = st[0] / n
    var = st[1] / n - mean * mean
    a = g * lax.rsqrt(var + 1e-5)
    t = be - mean * a
    return a.reshape(1, D), t.reshape(1, D)


def kernel(x, edge_index, edge_attr, batch, W_in, b_in, Wm1, bm1, gm1, bem1,
           Wm2, bm2, gm2, bem2, Wu1, bu1, gu1, beu1, Wu2, bu2, gu2, beu2,
           W_rm, b_rm, W_ro, b_ro, W_pm, b_pm, W_po, b_po):
    del batch, W_rm, b_rm, W_ro, b_ro

    src2 = edge_index[0].reshape(E // SUB, SUB)
    dst2 = edge_index[1].reshape(E // SUB, SUB)
    src2_a, src2_b = src2[:HA // SUB], src2[HA // SUB:]
    dst2_a, dst2_b = dst2[:HA // SUB], dst2[HA // SUB:]
    ea_a, ea_b = edge_attr[:HA], edge_attr[HA:]
    zero_init = jnp.zeros((NPAD // 16, D // 4), _f32)

    ngrid = (N // NB,)

    def edge1(hd, hs, ea, wa, wb, wc, b):
        ne = hd.shape[0]
        return pl.pallas_call(
            _edge1_body,
            grid=(ne // EB,),
            in_specs=[_rspec((EB, D)), _rspec((EB, D)), _rspec((EB, 16)),
                      _wspec((D, D)), _wspec((D, D)), _wspec((16, D)),
                      _wspec((1, D))],
            out_specs=[_rspec((EB, D)), _wspec((8, D))],
            out_shape=[jax.ShapeDtypeStruct((ne, D), _f32), _stats_shape()],
        )(hd, hs, ea, wa, wb, wc, b)

    def mlp2(z1, s1, t1, w, b, blk):
        n = z1.shape[0]
        return pl.pallas_call(
            _mlp2_body,
            grid=(n // blk,),
            in_specs=[_rspec((blk, D)), _wspec((1, D)), _wspec((1, D)),
                      _wspec((D, D)), _wspec((1, D))],
            out_specs=[_rspec((blk, D)), _wspec((8, D))],
            out_shape=[jax.ShapeDtypeStruct((n, D), _f32), _stats_shape()],
        )(z1, s1, t1, w, b)

    h = pl.pallas_call(
        _in_proj_body,
        grid=ngrid,
        in_specs=[_rspec((NB, 128)), _wspec((128, D)), _wspec((1, D))],
        out_specs=_rspec((NB, D)),
        out_shape=jax.ShapeDtypeStruct((N, D), _f32),
    )(x, W_in, b_in.reshape(1, D))

    for l in range(L):
        hd_a, hs_a = _gather_a(h, dst2_a, src2_a)
        hd_b, hs_b = _gather_b(h, dst2_b, src2_b)

        wa, wb, wc = Wm1[l, :D], Wm1[l, D:2 * D], Wm1[l, 2 * D:]
        bm = bm1[l].reshape(1, D)
        z1_a, st1_a = edge1(hd_a, hs_a, ea_a, wa, wb, wc, bm)
        z1_b, st1_b = edge1(hd_b, hs_b, ea_b, wa, wb, wc, bm)
        s1, t1 = _bn_coeffs(st1_a + st1_b, gm1[l], bem1[l], E)

        z2_a, st2_a = mlp2(z1_a, s1, t1, Wm2[l], bm2[l].reshape(1, D), EB)
        z2_b, st2_b = mlp2(z1_b, s1, t1, Wm2[l], bm2[l].reshape(1, D), EB)
        s2, t2 = _bn_coeffs(st2_a + st2_b, gm2[l], bem2[l], E)

        aggr_a = _scatter_a(z2_a, dst2_a, zero_init,
                            s2.reshape(4, 16), t2.reshape(4, 16))
        aggr_b = _scatter_b(z2_b, dst2_b, zero_init,
                            s2.reshape(4, 16), t2.reshape(4, 16))

        p = pl.pallas_call(
            _nodeA_body,
            grid=ngrid,
            in_specs=[_rspec((NB, D)), _wspec((D, D)), _wspec((1, D))],
            out_specs=_rspec((NB, D)),
            out_shape=jax.ShapeDtypeStruct((N, D), _f32),
        )(h, Wu1[l, :D], bu1[l].reshape(1, D))

        zu1, su1 = pl.pallas_call(
            _node1_body,
            grid=ngrid,
            in_specs=[_rspec((NB, D)), _rspec((NB, D)), _rspec((NB, D)),
                      _wspec((D, D))],
            out_specs=[_rspec((NB, D)), _wspec((8, D))],
            out_shape=[jax.ShapeDtypeStruct((N, D), _f32), _stats_shape()],
        )(p, aggr_a, aggr_b, Wu1[l, D:])
        a1, b1 = _bn_coeffs(su1, gu1[l], beu1[l], N)

        zu2, su2 = mlp2(zu1, a1, b1, Wu2[l], bu2[l].reshape(1, D), NB)
        a2, b2 = _bn_coeffs(su2, gu2[l], beu2[l], N)

        if l < L - 1:
            h = pl.pallas_call(
                _node3_body,
                grid=ngrid,
                in_specs=[_rspec((NB, D)), _wspec((1, D)), _wspec((1, D)),
                          _rspec((NB, D))],
                out_specs=_rspec((NB, D)),
                out_shape=jax.ShapeDtypeStruct((N, D), _f32),
            )(zu2, a2, b2, h)
        else:
            x_out = pl.pallas_call(
                _node3_head_body,
                grid=ngrid,
                in_specs=[_rspec((NB, D)), _wspec((1, D)), _wspec((1, D)),
                          _rspec((NB, D)), _wspec((D, D)), _wspec((1, D)),
                          _wspec((D, D)), _wspec((1, D))],
                out_specs=_rspec((NB, D)),
                out_shape=jax.ShapeDtypeStruct((N, D), _f32),
            )(zu2, a2, b2, h, W_pm, b_pm.reshape(1, D),
              W_po, b_po.reshape(1, D))
    return x_out


# speedup vs baseline: 1.1952x; 1.0537x over previous
import functools

import jax
import jax.numpy as jnp
from jax import lax
from jax.experimental import pallas as pl
from jax.experimental.pallas import tpu as pltpu
from jax.experimental.pallas import tpu_sc as plsc

N = 50000
E = 800000
L = 4
D = 64

EB = 4000
NB = 5000
NT = 32
SUB = 125
MAC = 1000
NPAD = 51200

HA = 416000
HB = 384000

_mesh = plsc.VectorSubcoreMesh(core_axis_name="c", subcore_axis_name="s")
_f32 = jnp.float32
_bf16 = jnp.bfloat16
_sc_params = pltpu.CompilerParams(use_tc_tiling_on_sc=False)


def _make_gather(ne):
    per_tile = ne // NT

    @functools.partial(
        pl.kernel,
        out_type=[jax.ShapeDtypeStruct((ne, D), _f32),
                  jax.ShapeDtypeStruct((ne, D), _f32)],
        mesh=_mesh,
        scratch_types=[pltpu.VMEM((8, SUB), jnp.int32),
                       pltpu.VMEM((MAC, D), _f32),
                       pltpu.SemaphoreType.DMA],
        compiler_params=_sc_params,
    )
    def gather(h_hbm, dst_hbm, src_hbm, hd_hbm, hs_hbm, idx_v, rows_v, sem):
        c = lax.axis_index("c")
        s = lax.axis_index("s")
        wid = s * 2 + c
        base = wid * per_tile

        @pl.loop(0, per_tile // MAC)
        def _(g):
            off = base + g * MAC
            row0 = pl.multiple_of(off // SUB, 8)
            for ind_hbm, out_hbm in ((dst_hbm, hd_hbm), (src_hbm, hs_hbm)):
                pltpu.sync_copy(ind_hbm.at[pl.ds(row0, 8)], idx_v)
                cps = [
                    pltpu.async_copy(h_hbm.at[idx_v.at[j]],
                                     rows_v.at[pl.ds(j * SUB, SUB)], sem)
                    for j in range(8)
                ]
                for cp in cps:
                    cp.wait()
                pltpu.sync_copy(rows_v, out_hbm.at[pl.ds(off, MAC)])

    return gather


_gather_a = _make_gather(HA)
_gather_b = _make_gather(HB)


def _make_scatter(ne):
    per_sub = ne // 16
    SMAC = 2000
    nsub = SMAC // SUB

    @functools.partial(
        pl.kernel,
        out_type=jax.ShapeDtypeStruct((NPAD, D), _f32),
        mesh=_mesh,
        scratch_types=[pltpu.VMEM((nsub, SUB), jnp.int32),
                       pltpu.VMEM((SMAC, D // 4), _f32),
                       pltpu.VMEM((4, 16), _f32),
                       pltpu.VMEM((4, 16), _f32),
                       pltpu.VMEM_SHARED((NPAD, D // 4), _f32),
                       pltpu.SemaphoreType.DMA],
        compiler_params=_sc_params,
    )
    def scatter(z_hbm, dst_hbm, zero_hbm, sc_hbm, sh_hbm, aggr_hbm,
                idx_v, mbuf_v, sc_v, sh_v, acc_sh, sem):
        c = lax.axis_index("c")
        s = lax.axis_index("s")
        rows_per_sub = NPAD // 16
        rbase = s * rows_per_sub
        ebase = s * per_sub

        pltpu.sync_copy(sc_hbm, sc_v)
        pltpu.sync_copy(sh_hbm, sh_v)

        for cc in range(2):
            colbase = c * (D // 2) + cc * (D // 4)
            kk = c * 2 + cc
            sv = sc_v[kk]
            tv = sh_v[kk]

            pltpu.sync_copy(zero_hbm, acc_sh.at[pl.ds(rbase, rows_per_sub)])
            plsc.subcore_barrier()

            @pl.loop(0, per_sub // SMAC)
            def _(g):
                off = ebase + g * SMAC
                row0 = pl.multiple_of(off // SUB, 8)
                pltpu.sync_copy(dst_hbm.at[pl.ds(row0, nsub)], idx_v)
                pltpu.sync_copy(z_hbm.at[pl.ds(off, SMAC),
                                         pl.ds(colbase, D // 4)], mbuf_v)

                @plsc.parallel_loop(0, SMAC, unroll=8)
                def _(r):
                    mbuf_v[r] = jnp.maximum(mbuf_v[r] * sv + tv, 0.0)

                cps = [
                    pltpu.async_copy(mbuf_v.at[pl.ds(j * SUB, SUB)],
                                     acc_sh.at[idx_v.at[j]], sem, add=True)
                    for j in range(nsub)
                ]
                for cp in cps:
                    cp.wait()

            plsc.subcore_barrier()
            pltpu.sync_copy(acc_sh.at[pl.ds(rbase, rows_per_sub)],
                            aggr_hbm.at[pl.ds(rbase, rows_per_sub),
                                        pl.ds(colbase, D // 4)])
            plsc.subcore_barrier()

    return scatter


_scatter_a = _make_scatter(HA)
_scatter_b = _make_scatter(HB)


def _rspec(blk):
    return pl.BlockSpec(blk, lambda i: (i, 0))


def _wspec(shape):
    return pl.BlockSpec(shape, lambda i: (0, 0))


def _accum_stats(st_ref, z):
    @pl.when(pl.program_id(0) == 0)
    def _():
        st_ref[...] = jnp.zeros_like(st_ref)

    st_ref[0:1, :] += jnp.sum(z, axis=0, keepdims=True)
    st_ref[1:2, :] += jnp.sum(z * z, axis=0, keepdims=True)


def _in_proj_body(x_ref, w_ref, b_ref, o_ref):
    o_ref[...] = (jnp.dot(x_ref[...], w_ref[...],
                          preferred_element_type=_f32) + b_ref[...])


def _bdot(a, w):
    return jnp.dot(a.astype(_bf16), w.astype(_bf16),
                   preferred_element_type=_f32)


def _edge1_body(hd, hs, ea, wa, wb, wc, b, z_ref, st_ref):
    z = (_bdot(hd[...], wa[...]) + _bdot(hs[...], wb[...])
         + _bdot(ea[...], wc[...]) + b[...])
    z_ref[...] = z.astype(z_ref.dtype)
    _accum_stats(st_ref, z)


def _mlp2_body(z1, s1, t1, w, b, z_ref, st_ref):
    a = jnp.maximum(z1[...].astype(_f32) * s1[...] + t1[...], 0.0)
    z = _bdot(a, w[...]) + b[...]
    z_ref[...] = z
    _accum_stats(st_ref, z)


def _nodeA_body(h, wa, b, o_ref):
    o_ref[...] = _bdot(h[...], wa[...]) + b[...]


def _node1_body(p, aga, agb, wb, z_ref, st_ref):
    z = p[...] + _bdot(aga[...] + agb[...], wb[...])
    z_ref[...] = z
    _accum_stats(st_ref, z)


def _node3_body(zu2, s, t, h, o_ref):
    o_ref[...] = h[...] + jnp.maximum(zu2[...] * s[...] + t[...], 0.0)


def _node3_head_body(zu2, s, t, h, wpm, bpm, wpo, bpo, o_ref):
    hn = h[...] + jnp.maximum(zu2[...] * s[...] + t[...], 0.0)
    hp = jnp.maximum(jnp.dot(hn, wpm[...], preferred_element_type=_f32)
                     + bpm[...], 0.0)
    o_ref[...] = (jnp.dot(hp, wpo[...], preferred_element_type=_f32)
                  + bpo[...])


def _stats_shape():
    return jax.ShapeDtypeStruct((8, D), _f32)


def _bn_coeffs(st, g, be, n):
    mean = st[0] / n
    var = st[1] / n - mean * mean
    a = g * lax.rsqrt(var + 1e-5)
    t = be - mean * a
    return a.reshape(1, D), t.reshape(1, D)


def kernel(x, edge_index, edge_attr, batch, W_in, b_in, Wm1, bm1, gm1, bem1,
           Wm2, bm2, gm2, bem2, Wu1, bu1, gu1, beu1, Wu2, bu2, gu2, beu2,
           W_rm, b_rm, W_ro, b_ro, W_pm, b_pm, W_po, b_po):
    del batch, W_rm, b_rm, W_ro, b_ro

    src2 = edge_index[0].reshape(E // SUB, SUB)
    dst2 = edge_index[1].reshape(E // SUB, SUB)
    src2_a, src2_b = src2[:HA // SUB], src2[HA // SUB:]
    dst2_a, dst2_b = dst2[:HA // SUB], dst2[HA // SUB:]
    ea_a, ea_b = edge_attr[:HA], edge_attr[HA:]
    zero_init = jnp.zeros((NPAD // 16, D // 4), _f32)

    ngrid = (N // NB,)

    def edge1(hd, hs, ea, wa, wb, wc, b):
        ne = hd.shape[0]
        return pl.pallas_call(
            _edge1_body,
            grid=(ne // EB,),
            in_specs=[_rspec((EB, D)), _rspec((EB, D)), _rspec((EB, 16)),
                      _wspec((D, D)), _wspec((D, D)), _wspec((16, D)),
                      _wspec((1, D))],
            out_specs=[_rspec((EB, D)), _wspec((8, D))],
            out_shape=[jax.ShapeDtypeStruct((ne, D), _bf16), _stats_shape()],
        )(hd, hs, ea, wa, wb, wc, b)

    def mlp2(z1, s1, t1, w, b, blk):
        n = z1.shape[0]
        return pl.pallas_call(
            _mlp2_body,
            grid=(n // blk,),
            in_specs=[_rspec((blk, D)), _wspec((1, D)), _wspec((1, D)),
                      _wspec((D, D)), _wspec((1, D))],
            out_specs=[_rspec((blk, D)), _wspec((8, D))],
            out_shape=[jax.ShapeDtypeStruct((n, D), _f32), _stats_shape()],
        )(z1, s1, t1, w, b)

    h = pl.pallas_call(
        _in_proj_body,
        grid=ngrid,
        in_specs=[_rspec((NB, 128)), _wspec((128, D)), _wspec((1, D))],
        out_specs=_rspec((NB, D)),
        out_shape=jax.ShapeDtypeStruct((N, D), _f32),
    )(x, W_in, b_in.reshape(1, D))

    for l in range(L):
        hd_a, hs_a = _gather_a(h, dst2_a, src2_a)
        hd_b, hs_b = _gather_b(h, dst2_b, src2_b)

        wa, wb, wc = Wm1[l, :D], Wm1[l, D:2 * D], Wm1[l, 2 * D:]
        bm = bm1[l].reshape(1, D)
        z1_a, st1_a = edge1(hd_a, hs_a, ea_a, wa, wb, wc, bm)
        z1_b, st1_b = edge1(hd_b, hs_b, ea_b, wa, wb, wc, bm)
        s1, t1 = _bn_coeffs(st1_a + st1_b, gm1[l], bem1[l], E)

        z2_a, st2_a = mlp2(z1_a, s1, t1, Wm2[l], bm2[l].reshape(1, D), EB)
        z2_b, st2_b = mlp2(z1_b, s1, t1, Wm2[l], bm2[l].reshape(1, D), EB)
        s2, t2 = _bn_coeffs(st2_a + st2_b, gm2[l], bem2[l], E)

        aggr_a = _scatter_a(z2_a, dst2_a, zero_init,
                            s2.reshape(4, 16), t2.reshape(4, 16))
        aggr_b = _scatter_b(z2_b, dst2_b, zero_init,
                            s2.reshape(4, 16), t2.reshape(4, 16))

        p = pl.pallas_call(
            _nodeA_body,
            grid=ngrid,
            in_specs=[_rspec((NB, D)), _wspec((D, D)), _wspec((1, D))],
            out_specs=_rspec((NB, D)),
            out_shape=jax.ShapeDtypeStruct((N, D), _f32),
        )(h, Wu1[l, :D], bu1[l].reshape(1, D))

        zu1, su1 = pl.pallas_call(
            _node1_body,
            grid=ngrid,
            in_specs=[_rspec((NB, D)), _rspec((NB, D)), _rspec((NB, D)),
                      _wspec((D, D))],
            out_specs=[_rspec((NB, D)), _wspec((8, D))],
            out_shape=[jax.ShapeDtypeStruct((N, D), _f32), _stats_shape()],
        )(p, aggr_a, aggr_b, Wu1[l, D:])
        a1, b1 = _bn_coeffs(su1, gu1[l], beu1[l], N)

        zu2, su2 = mlp2(zu1, a1, b1, Wu2[l], bu2[l].reshape(1, D), NB)
        a2, b2 = _bn_coeffs(su2, gu2[l], beu2[l], N)

        if l < L - 1:
            h = pl.pallas_call(
                _node3_body,
                grid=ngrid,
                in_specs=[_rspec((NB, D)), _wspec((1, D)), _wspec((1, D)),
                          _rspec((NB, D))],
                out_specs=_rspec((NB, D)),
                out_shape=jax.ShapeDtypeStruct((N, D), _f32),
            )(zu2, a2, b2, h)
        else:
            x_out = pl.pallas_call(
                _node3_head_body,
                grid=ngrid,
                in_specs=[_rspec((NB, D)), _wspec((1, D)), _wspec((1, D)),
                          _rspec((NB, D)), _wspec((D, D)), _wspec((1, D)),
                          _wspec((D, D)), _wspec((1, D))],
                out_specs=_rspec((NB, D)),
                out_shape=jax.ShapeDtypeStruct((N, D), _f32),
            )(zu2, a2, b2, h, W_pm, b_pm.reshape(1, D),
              W_po, b_po.reshape(1, D))
    return x_out
